# baseline (speedup 1.0000x reference)
import functools
import jax
import jax.numpy as jnp
from jax import lax
from jax.experimental import pallas as pl
from jax.experimental.pallas import tpu as pltpu
from jax.experimental.pallas import tpu_sc as plsc

V, D, B = 8192, 1024, 4096
_info = plsc.get_sparse_core_info()
NC, NS = _info.num_cores, _info.num_subcores
NW = NC * NS
B_PER_W = B // NW
CH = 32
NCHUNK = B_PER_W // CH


def _gather_kernel(table_hbm, idx_hbm, out_hbm, idx_v, sb0, sb1,
                   gsem0, gsem1, osem0, osem1):
    cid = lax.axis_index("c")
    sid = lax.axis_index("s")
    wid = sid * NC + cid
    base = wid * B_PER_W
    pltpu.sync_copy(idx_hbm.at[pl.ds(base, B_PER_W)], idx_v)

    sbufs = (sb0, sb1)
    gsems = (gsem0, gsem1)
    osems = (osem0, osem1)

    def issue_chunk(i, b):
        for jj in range(CH // 16):
            v = idx_v[pl.ds(i * CH + jj * 16, 16)]
            for k in range(16):
                r = v[k]
                pltpu.async_copy(
                    table_hbm.at[pl.ds(r, 1)],
                    sbufs[b].at[pl.ds(sid * CH + jj * 16 + k, 1)], gsems[b])

    def wait_chunk(b):
        pltpu.make_async_copy(
            table_hbm.at[pl.ds(0, CH)],
            sbufs[b].at[pl.ds(sid * CH, CH)], gsems[b]).wait()

    O = [None] * NCHUNK
    issue_chunk(0, 0)
    issue_chunk(1, 1)
    for i in range(NCHUNK):
        b = i % 2
        wait_chunk(b)
        if i >= 2:
            O[i - 2].wait()
        O[i] = pltpu.async_copy(
            sbufs[b].at[pl.ds(sid * CH, CH)],
            out_hbm.at[pl.ds(base + i * CH, CH)], osems[b])
        if i + 2 < NCHUNK:
            issue_chunk(i + 2, b)
    O[NCHUNK - 2].wait()
    O[NCHUNK - 1].wait()


@jax.jit
def _gather(table, idx):
    k = functools.partial(
        pl.kernel,
        mesh=plsc.VectorSubcoreMesh(core_axis_name="c", subcore_axis_name="s"),
        out_type=jax.ShapeDtypeStruct((B, D), jnp.float32),
        scratch_types=[
            pltpu.VMEM((B_PER_W,), jnp.int32),
            pltpu.MemorySpace.VMEM_SHARED((NS * 32, D), jnp.float32),
            pltpu.MemorySpace.VMEM_SHARED((NS * 32, D), jnp.float32),
            pltpu.SemaphoreType.DMA,
            pltpu.SemaphoreType.DMA,
            pltpu.SemaphoreType.DMA,
            pltpu.SemaphoreType.DMA,
        ],
    )(_gather_kernel)
    return k(table, idx)


def kernel(hidden_state, word_indices):
    table = hidden_state.reshape(V, D)
    idx = word_indices.astype(jnp.int32)
    out = _gather(table, idx)
    return out.reshape(1, B, D)


# speedup vs baseline: 1.0525x; 1.0525x over previous
import functools

import jax
import jax.numpy as jnp
from jax import lax
from jax.experimental import pallas as pl
from jax.experimental.pallas import tpu as pltpu
from jax.experimental.pallas import tpu_sc as plsc

V, D, B = 8192, 1024, 4096
_info = plsc.get_sparse_core_info()
NC, NS = _info.num_cores, _info.num_subcores
NW = NC * NS
B_PER_W = B // NW
NS_ROWS = 64
SCH = 32
DCHUNKS = (24, 24, 16)
DBUF = 24


def _gather_kernel(table_hbm, idx_hbm, out_hbm, idx_v, vb0, vb1, sb0, sb1,
                   gsem0, gsem1, dsem0, dsem1, osem, psem0, psem1):
    cid = lax.axis_index("c")
    sid = lax.axis_index("s")
    wid = sid * NC + cid
    base = wid * B_PER_W
    pltpu.sync_copy(idx_hbm.at[pl.ds(base, B_PER_W)],
                    idx_v.at[pl.ds(0, B_PER_W)])

    vbufs = (vb0, vb1)
    sbufs = (sb0, sb1)
    gsems = (gsem0, gsem1)
    dsems = (dsem0, dsem1)
    psems = (psem0, psem1)

    G = [pltpu.async_copy(
        table_hbm.at[idx_v.at[pl.ds(i * SCH, SCH)]], vbufs[i], gsems[i])
        for i in range(2)]

    offs = (0, DCHUNKS[0], DCHUNKS[0] + DCHUNKS[1])

    def issue_dchunk(j):
        b = j % 2
        n = DCHUNKS[j]
        for g in range((n + 15) // 16):
            v = idx_v[pl.ds(NS_ROWS + offs[j] + g * 16, 16)]
            for k in range(min(16, n - g * 16)):
                pltpu.async_copy(
                    table_hbm.at[pl.ds(v[k], 1)],
                    sbufs[b].at[pl.ds(sid * DBUF + g * 16 + k, 1)], dsems[b])

    def wait_dchunk(j):
        b = j % 2
        n = DCHUNKS[j]
        pltpu.make_async_copy(
            table_hbm.at[pl.ds(0, n)],
            sbufs[b].at[pl.ds(sid * DBUF, n)], dsems[b]).wait()

    def out_dchunk(j):
        b = j % 2
        n = DCHUNKS[j]
        return pltpu.async_copy(
            sbufs[b].at[pl.ds(sid * DBUF, n)],
            out_hbm.at[pl.ds(base + NS_ROWS + offs[j], n)], psems[b])

    issue_dchunk(0)
    issue_dchunk(1)

    O = [None, None]
    for i in range(2):
        G[i].wait()
        O[i] = pltpu.async_copy(
            vbufs[i], out_hbm.at[pl.ds(base + i * SCH, SCH)], osem)

    wait_dchunk(0)
    P0 = out_dchunk(0)
    wait_dchunk(1)
    P1 = out_dchunk(1)
    P0.wait()
    issue_dchunk(2)
    wait_dchunk(2)
    P2 = out_dchunk(2)

    O[0].wait()
    O[1].wait()
    P1.wait()
    P2.wait()


@jax.jit
def _gather(table, idx):
    k = functools.partial(
        pl.kernel,
        mesh=plsc.VectorSubcoreMesh(core_axis_name="c", subcore_axis_name="s"),
        out_type=jax.ShapeDtypeStruct((B, D), jnp.float32),
        scratch_types=[
            pltpu.VMEM((B_PER_W + 16,), jnp.int32),
            pltpu.VMEM((SCH, D), jnp.float32),
            pltpu.VMEM((SCH, D), jnp.float32),
            pltpu.MemorySpace.VMEM_SHARED((NS * DBUF, D), jnp.float32),
            pltpu.MemorySpace.VMEM_SHARED((NS * DBUF, D), jnp.float32),
            pltpu.SemaphoreType.DMA,
            pltpu.SemaphoreType.DMA,
            pltpu.SemaphoreType.DMA,
            pltpu.SemaphoreType.DMA,
            pltpu.SemaphoreType.DMA,
            pltpu.SemaphoreType.DMA,
            pltpu.SemaphoreType.DMA,
        ],
    )(_gather_kernel)
    return k(table, idx)


def kernel(hidden_state, word_indices):
    table = hidden_state.reshape(V, D)
    idx = word_indices.astype(jnp.int32)
    out = _gather(table, idx)
    return out.reshape(1, B, D)


# speedup vs baseline: 1.0694x; 1.0161x over previous
import functools

import jax
import jax.numpy as jnp
from jax import lax
from jax.experimental import pallas as pl
from jax.experimental.pallas import tpu as pltpu
from jax.experimental.pallas import tpu_sc as plsc

V, D, B = 8192, 1024, 4096
_info = plsc.get_sparse_core_info()
NC, NS = _info.num_cores, _info.num_subcores
NW = NC * NS
B_PER_W = B // NW
CH = 16
NCHUNK = B_PER_W // CH


def _gather_kernel(table_hbm, idx_hbm, out_hbm, idx_v, vb0, vb1, sb0, sb1,
                   gsem0, gsem1, csem0, csem1, osem0, osem1):
    cid = lax.axis_index("c")
    sid = lax.axis_index("s")
    wid = sid * NC + cid
    base = wid * B_PER_W
    pltpu.sync_copy(idx_hbm.at[pl.ds(base, B_PER_W)], idx_v)

    vbufs = (vb0, vb1)
    sbufs = (sb0, sb1)
    gsems = (gsem0, gsem1)
    csems = (csem0, csem1)
    osems = (osem0, osem1)

    def _reg(buf):
        return buf.at[pl.ds(sid * CH, CH)]

    G = [None] * NCHUNK
    C = [None] * NCHUNK
    O = [None] * NCHUNK
    G[0] = pltpu.async_copy(
        table_hbm.at[idx_v.at[pl.ds(0, CH)]], vbufs[0], gsems[0])
    G[1] = pltpu.async_copy(
        table_hbm.at[idx_v.at[pl.ds(CH, CH)]], vbufs[1], gsems[1])
    for i in range(NCHUNK):
        b = i % 2
        G[i].wait()
        if i >= 2:
            O[i - 2].wait()
        C[i] = pltpu.async_copy(vbufs[b], _reg(sbufs[b]), csems[b])
        C[i].wait()
        O[i] = pltpu.async_copy(
            _reg(sbufs[b]), out_hbm.at[pl.ds(base + i * CH, CH)], osems[b])
        if i + 2 < NCHUNK:
            G[i + 2] = pltpu.async_copy(
                table_hbm.at[idx_v.at[pl.ds((i + 2) * CH, CH)]],
                vbufs[b], gsems[b])
    O[NCHUNK - 2].wait()
    O[NCHUNK - 1].wait()


@jax.jit
def _gather(table, idx):
    k = functools.partial(
        pl.kernel,
        mesh=plsc.VectorSubcoreMesh(core_axis_name="c", subcore_axis_name="s"),
        out_type=jax.ShapeDtypeStruct((B, D), jnp.float32),
        scratch_types=[
            pltpu.VMEM((B_PER_W,), jnp.int32),
            pltpu.VMEM((CH, D), jnp.float32),
            pltpu.VMEM((CH, D), jnp.float32),
            pltpu.MemorySpace.VMEM_SHARED((NS * CH, D), jnp.float32),
            pltpu.MemorySpace.VMEM_SHARED((NS * CH, D), jnp.float32),
            pltpu.SemaphoreType.DMA,
            pltpu.SemaphoreType.DMA,
            pltpu.SemaphoreType.DMA,
            pltpu.SemaphoreType.DMA,
            pltpu.SemaphoreType.DMA,
            pltpu.SemaphoreType.DMA,
        ],
    )(_gather_kernel)
    return k(table, idx)


def kernel(hidden_state, word_indices):
    table = hidden_state.reshape(V, D)
    idx = word_indices.astype(jnp.int32)
    out = _gather(table, idx)
    return out.reshape(1, B, D)


# speedup vs baseline: 1.0916x; 1.0207x over previous
import functools

import jax
import jax.numpy as jnp
from jax import lax
from jax.experimental import pallas as pl
from jax.experimental.pallas import tpu as pltpu
from jax.experimental.pallas import tpu_sc as plsc

V, D, B = 8192, 1024, 4096
_info = plsc.get_sparse_core_info()
NC, NS = _info.num_cores, _info.num_subcores
NW = NC * NS
B_PER_W = B // NW
CH = 16
NCHUNK = B_PER_W // CH


NB = 3


def _gather_kernel(table_hbm, idx_hbm, out_hbm, idx_v, vb0, vb1, vb2,
                   sb0, sb1, sb2, gsem0, gsem1, gsem2, csem0, csem1, csem2,
                   osem0, osem1, osem2):
    cid = lax.axis_index("c")
    sid = lax.axis_index("s")
    wid = sid * NC + cid
    base = wid * B_PER_W
    pltpu.sync_copy(idx_hbm.at[pl.ds(base, B_PER_W)], idx_v)

    vbufs = (vb0, vb1, vb2)
    sbufs = (sb0, sb1, sb2)
    gsems = (gsem0, gsem1, gsem2)
    csems = (csem0, csem1, csem2)
    osems = (osem0, osem1, osem2)

    def _reg(buf):
        return buf.at[pl.ds(sid * CH, CH)]

    G = [None] * NCHUNK
    C = [None] * NCHUNK
    O = [None] * NCHUNK
    for i in range(NB):
        G[i] = pltpu.async_copy(
            table_hbm.at[idx_v.at[pl.ds(i * CH, CH)]], vbufs[i], gsems[i])
    for i in range(NCHUNK):
        b = i % NB
        G[i].wait()
        if i >= NB:
            O[i - NB].wait()
        C[i] = pltpu.async_copy(vbufs[b], _reg(sbufs[b]), csems[b])
        C[i].wait()
        O[i] = pltpu.async_copy(
            _reg(sbufs[b]), out_hbm.at[pl.ds(base + i * CH, CH)], osems[b])
        if i + NB < NCHUNK:
            G[i + NB] = pltpu.async_copy(
                table_hbm.at[idx_v.at[pl.ds((i + NB) * CH, CH)]],
                vbufs[b], gsems[b])
    for i in range(NCHUNK - NB, NCHUNK):
        O[i].wait()


@jax.jit
def _gather(table, idx):
    k = functools.partial(
        pl.kernel,
        mesh=plsc.VectorSubcoreMesh(core_axis_name="c", subcore_axis_name="s"),
        out_type=jax.ShapeDtypeStruct((B, D), jnp.float32),
        scratch_types=[
            pltpu.VMEM((B_PER_W,), jnp.int32),
            pltpu.VMEM((CH, D), jnp.float32),
            pltpu.VMEM((CH, D), jnp.float32),
            pltpu.VMEM((CH, D), jnp.float32),
            pltpu.MemorySpace.VMEM_SHARED((NS * CH, D), jnp.float32),
            pltpu.MemorySpace.VMEM_SHARED((NS * CH, D), jnp.float32),
            pltpu.MemorySpace.VMEM_SHARED((NS * CH, D), jnp.float32),
            pltpu.SemaphoreType.DMA,
            pltpu.SemaphoreType.DMA,
            pltpu.SemaphoreType.DMA,
            pltpu.SemaphoreType.DMA,
            pltpu.SemaphoreType.DMA,
            pltpu.SemaphoreType.DMA,
            pltpu.SemaphoreType.DMA,
            pltpu.SemaphoreType.DMA,
            pltpu.SemaphoreType.DMA,
        ],
    )(_gather_kernel)
    return k(table, idx)


def kernel(hidden_state, word_indices):
    table = hidden_state.reshape(V, D)
    idx = word_indices.astype(jnp.int32)
    out = _gather(table, idx)
    return out.reshape(1, B, D)


# speedup vs baseline: 1.1112x; 1.0180x over previous
import functools

import jax
import jax.numpy as jnp
from jax import lax
from jax.experimental import pallas as pl
from jax.experimental.pallas import tpu as pltpu
from jax.experimental.pallas import tpu_sc as plsc

V, D, B = 8192, 1024, 4096
_info = plsc.get_sparse_core_info()
NC, NS = _info.num_cores, _info.num_subcores
NW = NC * NS
B_PER_W = B // NW
CH = 8
NCHUNK = B_PER_W // CH


NB = 6


def _gather_kernel(table_hbm, idx_hbm, out_hbm, idx_v, vb0, vb1, vb2, vb3,
                   vb4, vb5, sb0, sb1, sb2, sb3, sb4, sb5, gsem0, gsem1,
                   gsem2, gsem3, gsem4, gsem5, csem0, csem1, csem2, csem3,
                   csem4, csem5, osem0, osem1, osem2, osem3, osem4, osem5):
    cid = lax.axis_index("c")
    sid = lax.axis_index("s")
    wid = sid * NC + cid
    base = wid * B_PER_W
    pltpu.sync_copy(idx_hbm.at[pl.ds(base, B_PER_W)], idx_v)

    vbufs = (vb0, vb1, vb2, vb3, vb4, vb5)
    sbufs = (sb0, sb1, sb2, sb3, sb4, sb5)
    gsems = (gsem0, gsem1, gsem2, gsem3, gsem4, gsem5)
    csems = (csem0, csem1, csem2, csem3, csem4, csem5)
    osems = (osem0, osem1, osem2, osem3, osem4, osem5)

    def _reg(buf):
        return buf.at[pl.ds(sid * CH, CH)]

    G = [None] * NCHUNK
    C = [None] * NCHUNK
    O = [None] * NCHUNK
    for i in range(NB):
        G[i] = pltpu.async_copy(
            table_hbm.at[idx_v.at[pl.ds(i * CH, CH)]], vbufs[i], gsems[i])
    for i in range(NCHUNK):
        b = i % NB
        G[i].wait()
        if i >= NB:
            O[i - NB].wait()
        C[i] = pltpu.async_copy(vbufs[b], _reg(sbufs[b]), csems[b])
        C[i].wait()
        O[i] = pltpu.async_copy(
            _reg(sbufs[b]), out_hbm.at[pl.ds(base + i * CH, CH)], osems[b])
        if i + NB < NCHUNK:
            G[i + NB] = pltpu.async_copy(
                table_hbm.at[idx_v.at[pl.ds((i + NB) * CH, CH)]],
                vbufs[b], gsems[b])
    for i in range(NCHUNK - NB, NCHUNK):
        O[i].wait()


@jax.jit
def _gather(table, idx):
    k = functools.partial(
        pl.kernel,
        mesh=plsc.VectorSubcoreMesh(core_axis_name="c", subcore_axis_name="s"),
        out_type=jax.ShapeDtypeStruct((B, D), jnp.float32),
        scratch_types=[
            pltpu.VMEM((B_PER_W,), jnp.int32),
            pltpu.VMEM((CH, D), jnp.float32),
            pltpu.VMEM((CH, D), jnp.float32),
            pltpu.VMEM((CH, D), jnp.float32),
            pltpu.VMEM((CH, D), jnp.float32),
            pltpu.VMEM((CH, D), jnp.float32),
            pltpu.VMEM((CH, D), jnp.float32),
            pltpu.MemorySpace.VMEM_SHARED((NS * CH, D), jnp.float32),
            pltpu.MemorySpace.VMEM_SHARED((NS * CH, D), jnp.float32),
            pltpu.MemorySpace.VMEM_SHARED((NS * CH, D), jnp.float32),
            pltpu.MemorySpace.VMEM_SHARED((NS * CH, D), jnp.float32),
            pltpu.MemorySpace.VMEM_SHARED((NS * CH, D), jnp.float32),
            pltpu.MemorySpace.VMEM_SHARED((NS * CH, D), jnp.float32),
        ] + [pltpu.SemaphoreType.DMA] * 18,
    )(_gather_kernel)
    return k(table, idx)


def kernel(hidden_state, word_indices):
    table = hidden_state.reshape(V, D)
    idx = word_indices.astype(jnp.int32)
    out = _gather(table, idx)
    return out.reshape(1, B, D)


# speedup vs baseline: 1.1313x; 1.0181x over previous
import functools

import jax
import jax.numpy as jnp
from jax import lax
from jax.experimental import pallas as pl
from jax.experimental.pallas import tpu as pltpu
from jax.experimental.pallas import tpu_sc as plsc

V, D, B = 8192, 1024, 4096
_info = plsc.get_sparse_core_info()
NC, NS = _info.num_cores, _info.num_subcores
NW = NC * NS
B_PER_W = B // NW
CH = 8
NCHUNK = B_PER_W // CH
NB = 12


def _gather_kernel(table_hbm, idx_hbm, out_hbm, idx_v, *rest):
    vbufs = rest[:NB]
    gsems = rest[NB:2 * NB]
    osems = rest[2 * NB:3 * NB]
    cid = lax.axis_index("c")
    sid = lax.axis_index("s")
    wid = sid * NC + cid
    base = wid * B_PER_W
    pltpu.sync_copy(idx_hbm.at[pl.ds(base, B_PER_W)], idx_v)

    G = [None] * NCHUNK
    O = [None] * NCHUNK
    for i in range(NB):
        G[i] = pltpu.async_copy(
            table_hbm.at[idx_v.at[pl.ds(i * CH, CH)]], vbufs[i], gsems[i])
    LAG = 4
    for i in range(NCHUNK):
        b = i % NB
        G[i].wait()
        O[i] = pltpu.async_copy(
            vbufs[b], out_hbm.at[pl.ds(base + i * CH, CH)], osems[b])
        j = i - LAG
        if j >= 0 and j + NB < NCHUNK:
            O[j].wait()
            G[j + NB] = pltpu.async_copy(
                table_hbm.at[idx_v.at[pl.ds((j + NB) * CH, CH)]],
                vbufs[j % NB], gsems[j % NB])
    for i in range(NCHUNK):
        if i + NB >= NCHUNK or i > NCHUNK - 1 - LAG:
            O[i].wait()


@jax.jit
def _gather(table, idx):
    k = functools.partial(
        pl.kernel,
        mesh=plsc.VectorSubcoreMesh(core_axis_name="c", subcore_axis_name="s"),
        out_type=jax.ShapeDtypeStruct((B, D), jnp.float32),
        scratch_types=[pltpu.VMEM((B_PER_W,), jnp.int32)]
        + [pltpu.VMEM((CH, D), jnp.float32)] * NB
        + [pltpu.SemaphoreType.DMA] * (2 * NB),
    )(_gather_kernel)
    return k(table, idx)


def kernel(hidden_state, word_indices):
    table = hidden_state.reshape(V, D)
    idx = word_indices.astype(jnp.int32)
    out = _gather(table, idx)
    return out.reshape(1, B, D)


# speedup vs baseline: 1.1326x; 1.0011x over previous
import functools

import jax
import jax.numpy as jnp
from jax import lax
from jax.experimental import pallas as pl
from jax.experimental.pallas import tpu as pltpu
from jax.experimental.pallas import tpu_sc as plsc

V, D, B = 8192, 1024, 4096
_info = plsc.get_sparse_core_info()
NC, NS = _info.num_cores, _info.num_subcores
NW = NC * NS
B_PER_W = B // NW
CH = 16
NCHUNK = B_PER_W // CH
NB = 7


def _gather_kernel(table_hbm, idx_hbm, out_hbm, idx_v, *rest):
    vbufs = rest[:NB]
    gsems = rest[NB:2 * NB]
    osems = rest[2 * NB:3 * NB]
    cid = lax.axis_index("c")
    sid = lax.axis_index("s")
    wid = sid * NC + cid
    base = wid * B_PER_W
    pltpu.sync_copy(idx_hbm.at[pl.ds(base, B_PER_W)], idx_v)

    G = [None] * NCHUNK
    O = [None] * NCHUNK
    for i in range(NB):
        G[i] = pltpu.async_copy(
            table_hbm.at[idx_v.at[pl.ds(i * CH, CH)]], vbufs[i], gsems[i])
    LAG = 1
    for i in range(NCHUNK):
        b = i % NB
        G[i].wait()
        O[i] = pltpu.async_copy(
            vbufs[b], out_hbm.at[pl.ds(base + i * CH, CH)], osems[b])
        j = i - LAG
        if j >= 0 and j + NB < NCHUNK:
            O[j].wait()
            G[j + NB] = pltpu.async_copy(
                table_hbm.at[idx_v.at[pl.ds((j + NB) * CH, CH)]],
                vbufs[j % NB], gsems[j % NB])
    for i in range(NCHUNK):
        if i + NB >= NCHUNK or i > NCHUNK - 1 - LAG:
            O[i].wait()


@jax.jit
def _gather(table, idx):
    k = functools.partial(
        pl.kernel,
        mesh=plsc.VectorSubcoreMesh(core_axis_name="c", subcore_axis_name="s"),
        out_type=jax.ShapeDtypeStruct((B, D), jnp.float32),
        scratch_types=[pltpu.VMEM((B_PER_W,), jnp.int32)]
        + [pltpu.VMEM((CH, D), jnp.float32)] * NB
        + [pltpu.SemaphoreType.DMA] * (2 * NB),
    )(_gather_kernel)
    return k(table, idx)


def kernel(hidden_state, word_indices):
    table = hidden_state.reshape(V, D)
    idx = word_indices.astype(jnp.int32)
    out = _gather(table, idx)
    return out.reshape(1, B, D)


# speedup vs baseline: 1.1445x; 1.0104x over previous
import functools

import jax
import jax.numpy as jnp
from jax import lax
from jax.experimental import pallas as pl
from jax.experimental.pallas import tpu as pltpu
from jax.experimental.pallas import tpu_sc as plsc

V, D, B = 8192, 1024, 4096
_info = plsc.get_sparse_core_info()
NC, NS = _info.num_cores, _info.num_subcores
NW = NC * NS
B_PER_W = B // NW
CH = 8
NCHUNK = B_PER_W // CH
NB = 15


def _gather_kernel(table_hbm, idx_hbm, out_hbm, idx_v, *rest):
    vbufs = rest[:NB]
    gsems = rest[NB:2 * NB]
    osems = rest[2 * NB:3 * NB]
    cid = lax.axis_index("c")
    sid = lax.axis_index("s")
    wid = sid * NC + cid
    base = wid * B_PER_W
    pltpu.sync_copy(idx_hbm.at[pl.ds(base, B_PER_W)], idx_v)

    G = [None] * NCHUNK
    O = [None] * NCHUNK
    for i in range(NB):
        G[i] = pltpu.async_copy(
            table_hbm.at[idx_v.at[pl.ds(i * CH, CH)]], vbufs[i], gsems[i])
    LAG = 2
    for i in range(NCHUNK):
        b = i % NB
        G[i].wait()
        O[i] = pltpu.async_copy(
            vbufs[b], out_hbm.at[pl.ds(base + i * CH, CH)], osems[b])
        j = i - LAG
        if j >= 0 and j + NB < NCHUNK:
            O[j].wait()
            G[j + NB] = pltpu.async_copy(
                table_hbm.at[idx_v.at[pl.ds((j + NB) * CH, CH)]],
                vbufs[j % NB], gsems[j % NB])
    for i in range(NCHUNK):
        if i + NB >= NCHUNK or i > NCHUNK - 1 - LAG:
            O[i].wait()


@jax.jit
def _gather(table, idx):
    k = functools.partial(
        pl.kernel,
        mesh=plsc.VectorSubcoreMesh(core_axis_name="c", subcore_axis_name="s"),
        out_type=jax.ShapeDtypeStruct((B, D), jnp.float32),
        scratch_types=[pltpu.VMEM((B_PER_W,), jnp.int32)]
        + [pltpu.VMEM((CH, D), jnp.float32)] * NB
        + [pltpu.SemaphoreType.DMA] * (2 * NB),
    )(_gather_kernel)
    return k(table, idx)


def kernel(hidden_state, word_indices):
    table = hidden_state.reshape(V, D)
    idx = word_indices.astype(jnp.int32)
    out = _gather(table, idx)
    return out.reshape(1, B, D)
